# baseline (speedup 1.0000x reference)
import functools

import jax
import jax.numpy as jnp
from jax.experimental import pallas as pl
from jax.experimental.pallas import tpu as pltpu

B = 16
MAX_T = 4096
S_DIM = 128
NA = 4
BT = 256
NT = MAX_T // BT


def _tc_body(lens_ref, s_ref, a_ref, wp_ref, bias_ref, out_ref):
    b = pl.program_id(0)
    t = pl.program_id(1)
    x = s_ref[0]
    z = jnp.dot(x, wp_ref[...], preferred_element_type=jnp.float32)
    z = z + bias_ref[...]
    lane = jax.lax.broadcasted_iota(jnp.int32, (BT, 128), 1)
    valid = lane < NA
    neg_inf = jnp.float32(-jnp.inf)
    m = jnp.max(jnp.where(valid, z, neg_inf), axis=1, keepdims=True)
    e = jnp.sum(jnp.exp(jnp.where(valid, z - m, neg_inf)), axis=1,
                keepdims=True)
    lse = m + jnp.log(e)
    a = a_ref[0, 0, 0]
    taken = jnp.sum(jnp.where(lane == a[:, None], z, 0.0), axis=1,
                    keepdims=True)
    row = jax.lax.broadcasted_iota(jnp.int32, (BT, 1), 0)
    live = (t * BT + row) < lens_ref[b]
    partial = jnp.sum(jnp.where(live, lse - taken, 0.0)).reshape(1, 1)
    first = jnp.logical_and(b == 0, t == 0)
    acc = jnp.where(first, jnp.zeros((1, 1), jnp.float32), out_ref[...])
    out_ref[...] = acc + partial


@jax.jit
def _tc_loss(s, actions4, lengths, wp, bias_pad):
    grid_spec = pltpu.PrefetchScalarGridSpec(
        num_scalar_prefetch=1,
        grid=(B, NT),
        in_specs=[
            pl.BlockSpec(
                (1, BT, S_DIM),
                lambda b, t, lens: (b, jnp.minimum(t, (lens[b] - 1) // BT), 0)),
            pl.BlockSpec(
                (1, 1, 1, BT),
                lambda b, t, lens: (b, jnp.minimum(t, (lens[b] - 1) // BT),
                                    0, 0)),
            pl.BlockSpec((S_DIM, 128), lambda b, t, lens: (0, 0)),
            pl.BlockSpec((1, 128), lambda b, t, lens: (0, 0)),
        ],
        out_specs=pl.BlockSpec((1, 1), lambda b, t, lens: (0, 0)),
    )
    out = pl.pallas_call(
        _tc_body,
        grid_spec=grid_spec,
        out_shape=jax.ShapeDtypeStruct((1, 1), jnp.float32),
    )(lengths, s, actions4, wp, bias_pad)
    return out[0, 0]


def kernel(s_i_batch, actions_batch, lengths, W, bias, W_stop, W_start):
    del W_stop, W_start
    wp = jnp.zeros((S_DIM, 128), jnp.float32).at[:, :NA].set(W[:, :NA])
    bias_pad = jnp.zeros((1, 128), jnp.float32).at[0, :NA].set(bias[:NA])
    actions4 = actions_batch.astype(jnp.int32).reshape(B, NT, 1, BT)
    return _tc_loss(s_i_batch, actions4, lengths.astype(jnp.int32), wp,
                    bias_pad)


# speedup vs baseline: 1.3081x; 1.3081x over previous
import jax
import jax.numpy as jnp
from jax.experimental import pallas as pl
from jax.experimental.pallas import tpu as pltpu

B = 16
MAX_T = 4096
S_DIM = 128
NA = 4
BT = 512
NT = MAX_T // BT
MAXG = B * NT


def _tc_body(nlive_ref, bid_ref, tid_ref, s_ref, a_ref, wp_ref, bias_ref,
             out_ref):
    i = pl.program_id(0)
    x = s_ref[0]
    z = jnp.dot(x, wp_ref[...], preferred_element_type=jnp.float32)
    zt = z.T[:NA] + bias_ref[...]
    m = jnp.max(zt, axis=0, keepdims=True)
    e = jnp.sum(jnp.exp(zt - m), axis=0, keepdims=True)
    lse = m + jnp.log(e)
    a = a_ref[0, 0]
    taken = jnp.where(a == 0, zt[0:1], 0.0)
    for j in range(1, NA):
        taken += jnp.where(a == j, zt[j:j + 1], 0.0)
    lane = jax.lax.broadcasted_iota(jnp.int32, (1, BT), 1)
    live = lane < nlive_ref[i]
    contrib = jnp.where(live, lse - taken, 0.0)
    acc = jnp.where(i == 0, jnp.zeros((1, BT), jnp.float32), out_ref[...])
    out_ref[...] = acc + contrib


@jax.jit
def _tc_loss(s, actions4, lengths, wp, bias_col):
    lengths = lengths.astype(jnp.int32)
    nblk = (lengths + BT - 1) // BT
    g = jnp.sum(nblk)
    cum = jnp.cumsum(nblk)
    flat = jnp.arange(MAXG, dtype=jnp.int32)
    bid = jnp.searchsorted(cum, flat, side="right").astype(jnp.int32)
    bidc = jnp.minimum(bid, B - 1)
    tid = flat - jnp.where(bid > 0, cum[jnp.maximum(bid - 1, 0)], 0)
    nlive = jnp.clip(lengths[bidc] - tid * BT, 0, BT)

    grid_spec = pltpu.PrefetchScalarGridSpec(
        num_scalar_prefetch=3,
        grid=(g,),
        in_specs=[
            pl.BlockSpec(
                (1, BT, S_DIM),
                lambda i, nlive, bid, tid: (bid[i], tid[i], 0)),
            pl.BlockSpec(
                (1, 1, 1, BT),
                lambda i, nlive, bid, tid: (bid[i], tid[i], 0, 0)),
            pl.BlockSpec((S_DIM, 8), lambda i, nlive, bid, tid: (0, 0)),
            pl.BlockSpec((NA, 1), lambda i, nlive, bid, tid: (0, 0)),
        ],
        out_specs=pl.BlockSpec((1, BT), lambda i, nlive, bid, tid: (0, 0)),
    )
    out = pl.pallas_call(
        _tc_body,
        grid_spec=grid_spec,
        out_shape=jax.ShapeDtypeStruct((1, BT), jnp.float32),
    )(nlive, bidc, tid, s, actions4, wp, bias_col)
    return jnp.sum(out)


def kernel(s_i_batch, actions_batch, lengths, W, bias, W_stop, W_start):
    del W_stop, W_start
    wp = jnp.zeros((S_DIM, 8), jnp.float32).at[:, :NA].set(W[:, :NA])
    bias_col = bias[:NA].reshape(NA, 1)
    actions4 = actions_batch.astype(jnp.int32).reshape(B, NT, 1, BT)
    return _tc_loss(s_i_batch, actions4, lengths, wp, bias_col)


# speedup vs baseline: 1.5114x; 1.1554x over previous
import jax
import jax.numpy as jnp
from jax.experimental import pallas as pl
from jax.experimental.pallas import tpu as pltpu

B = 16
MAX_T = 4096
S_DIM = 128
NA = 4
BT = 512
NT = MAX_T // BT
MAXG = B * NT
NBUF = 4


def _body(g_ref, nlive_ref, bid_ref, tid_ref, s_ref, a_ref, wp_ref, bias_ref,
          out_ref, sbuf, abuf, sem_s, sem_a):
    g = g_ref[0]

    def copies(i, slot):
        b = bid_ref[i]
        t = tid_ref[i]
        c1 = pltpu.make_async_copy(
            s_ref.at[b, pl.ds(t * BT, BT), :], sbuf.at[slot], sem_s.at[slot])
        c2 = pltpu.make_async_copy(
            a_ref.at[b, t], abuf.at[slot], sem_a.at[slot])
        return c1, c2

    for k in range(NBUF - 1):
        c1, c2 = copies(k, k)
        c1.start()
        c2.start()

    def step(i, acc):
        slot = jax.lax.rem(i, NBUF)

        @pl.when(i + NBUF - 1 < g)
        def _():
            c1, c2 = copies(i + NBUF - 1, jax.lax.rem(i + NBUF - 1, NBUF))
            c1.start()
            c2.start()

        c1, c2 = copies(i, slot)
        c1.wait()
        c2.wait()

        x = sbuf[slot]
        z = jnp.dot(x, wp_ref[...], preferred_element_type=jnp.float32)
        zt = z.T[:NA] + bias_ref[...]
        m = jnp.max(zt, axis=0, keepdims=True)
        e = jnp.sum(jnp.exp(zt - m), axis=0, keepdims=True)
        lse = m + jnp.log(e)
        a = abuf[slot]
        taken = jnp.where(a == 0, zt[0:1], 0.0)
        for j in range(1, NA):
            taken += jnp.where(a == j, zt[j:j + 1], 0.0)
        lane = jax.lax.broadcasted_iota(jnp.int32, (1, BT), 1)
        live = lane < nlive_ref[i]
        return acc + jnp.where(live, lse - taken, 0.0)

    out_ref[...] = jax.lax.fori_loop(
        0, g, step, jnp.zeros((1, BT), jnp.float32))


@jax.jit
def _tc_loss(s, actions4, lengths, wp, bias_col):
    lengths = lengths.astype(jnp.int32)
    nblk = (lengths + BT - 1) // BT
    g = jnp.sum(nblk).reshape(1)
    cum = jnp.cumsum(nblk)
    flat = jnp.arange(MAXG, dtype=jnp.int32)
    bid = jnp.searchsorted(cum, flat, side="right").astype(jnp.int32)
    bidc = jnp.minimum(bid, B - 1)
    tid = flat - jnp.where(bid > 0, cum[jnp.maximum(bid - 1, 0)], 0)
    nlive = jnp.clip(lengths[bidc] - tid * BT, 0, BT)

    grid_spec = pltpu.PrefetchScalarGridSpec(
        num_scalar_prefetch=4,
        grid=(1,),
        in_specs=[
            pl.BlockSpec(memory_space=pltpu.MemorySpace.HBM),
            pl.BlockSpec(memory_space=pltpu.MemorySpace.HBM),
            pl.BlockSpec((S_DIM, 8), lambda i, *_: (0, 0)),
            pl.BlockSpec((NA, 1), lambda i, *_: (0, 0)),
        ],
        out_specs=pl.BlockSpec((1, BT), lambda i, *_: (0, 0)),
        scratch_shapes=[
            pltpu.VMEM((NBUF, BT, S_DIM), jnp.float32),
            pltpu.VMEM((NBUF, 1, BT), jnp.int32),
            pltpu.SemaphoreType.DMA((NBUF,)),
            pltpu.SemaphoreType.DMA((NBUF,)),
        ],
    )
    out = pl.pallas_call(
        _body,
        grid_spec=grid_spec,
        out_shape=jax.ShapeDtypeStruct((1, BT), jnp.float32),
    )(g, nlive, bidc, tid, s, actions4, wp, bias_col)
    return jnp.sum(out)


def kernel(s_i_batch, actions_batch, lengths, W, bias, W_stop, W_start):
    del W_stop, W_start
    wp = jnp.zeros((S_DIM, 8), jnp.float32).at[:, :NA].set(W[:, :NA])
    bias_col = bias[:NA].reshape(NA, 1)
    actions4 = actions_batch.astype(jnp.int32).reshape(B, NT, 1, BT)
    return _tc_loss(s_i_batch, actions4, lengths, wp, bias_col)


# speedup vs baseline: 1.5261x; 1.0097x over previous
import jax
import jax.numpy as jnp
from jax.experimental import pallas as pl
from jax.experimental.pallas import tpu as pltpu

B = 16
MAX_T = 4096
S_DIM = 128
NA = 4
BT = 512
NT = MAX_T // BT
MAXG = B * NT
NBUF = 8


def _body(g_ref, nlive_ref, bid_ref, tid_ref, s_ref, a_ref, wp_ref, bias_ref,
          out_ref, sbuf, abuf, sem_s, sem_a):
    g = g_ref[0]

    def copies(i, slot):
        b = bid_ref[i]
        t = tid_ref[i]
        c1 = pltpu.make_async_copy(
            s_ref.at[b, pl.ds(t * BT, BT), :], sbuf.at[slot], sem_s.at[slot])
        c2 = pltpu.make_async_copy(
            a_ref.at[b, t], abuf.at[slot], sem_a.at[slot])
        return c1, c2

    for k in range(NBUF - 1):
        c1, c2 = copies(k, k)
        c1.start()
        c2.start()

    def step(i, acc):
        slot = jax.lax.rem(i, NBUF)

        @pl.when(i + NBUF - 1 < g)
        def _():
            c1, c2 = copies(i + NBUF - 1, jax.lax.rem(i + NBUF - 1, NBUF))
            c1.start()
            c2.start()

        c1, c2 = copies(i, slot)
        c1.wait()
        c2.wait()

        x = sbuf[slot]
        z = jnp.dot(x, wp_ref[...], preferred_element_type=jnp.float32)
        zt = z.T[:NA] + bias_ref[...]
        m = jnp.max(zt, axis=0, keepdims=True)
        e = jnp.sum(jnp.exp(zt - m), axis=0, keepdims=True)
        lse = m + jnp.log(e)
        a = abuf[slot]
        taken = jnp.where(a == 0, zt[0:1], 0.0)
        for j in range(1, NA):
            taken += jnp.where(a == j, zt[j:j + 1], 0.0)
        lane = jax.lax.broadcasted_iota(jnp.int32, (1, BT), 1)
        live = lane < nlive_ref[i]
        return acc + jnp.where(live, lse - taken, 0.0)

    out_ref[...] = jax.lax.fori_loop(
        0, g, step, jnp.zeros((1, BT), jnp.float32))


@jax.jit
def _tc_loss(s, actions4, lengths, wp, bias_col):
    lengths = lengths.astype(jnp.int32)
    nblk = (lengths + BT - 1) // BT
    g = jnp.sum(nblk).reshape(1)
    cum = jnp.cumsum(nblk)
    flat = jnp.arange(MAXG, dtype=jnp.int32)
    bid = jnp.searchsorted(cum, flat, side="right").astype(jnp.int32)
    bidc = jnp.minimum(bid, B - 1)
    tid = flat - jnp.where(bid > 0, cum[jnp.maximum(bid - 1, 0)], 0)
    nlive = jnp.clip(lengths[bidc] - tid * BT, 0, BT)

    grid_spec = pltpu.PrefetchScalarGridSpec(
        num_scalar_prefetch=4,
        grid=(1,),
        in_specs=[
            pl.BlockSpec(memory_space=pltpu.MemorySpace.HBM),
            pl.BlockSpec(memory_space=pltpu.MemorySpace.HBM),
            pl.BlockSpec((S_DIM, 8), lambda i, *_: (0, 0)),
            pl.BlockSpec((NA, 1), lambda i, *_: (0, 0)),
        ],
        out_specs=pl.BlockSpec((1, BT), lambda i, *_: (0, 0)),
        scratch_shapes=[
            pltpu.VMEM((NBUF, BT, S_DIM), jnp.float32),
            pltpu.VMEM((NBUF, 1, BT), jnp.int32),
            pltpu.SemaphoreType.DMA((NBUF,)),
            pltpu.SemaphoreType.DMA((NBUF,)),
        ],
    )
    out = pl.pallas_call(
        _body,
        grid_spec=grid_spec,
        out_shape=jax.ShapeDtypeStruct((1, BT), jnp.float32),
    )(g, nlive, bidc, tid, s, actions4, wp, bias_col)
    return jnp.sum(out)


def kernel(s_i_batch, actions_batch, lengths, W, bias, W_stop, W_start):
    del W_stop, W_start
    wp = jnp.zeros((S_DIM, 8), jnp.float32).at[:, :NA].set(W[:, :NA])
    bias_col = bias[:NA].reshape(NA, 1)
    actions4 = actions_batch.astype(jnp.int32).reshape(B, NT, 1, BT)
    return _tc_loss(s_i_batch, actions4, lengths, wp, bias_col)


# speedup vs baseline: 3.4921x; 2.2883x over previous
import jax
import jax.numpy as jnp
from jax.experimental import pallas as pl
from jax.experimental.pallas import tpu as pltpu

B = 16
MAX_T = 4096
S_DIM = 128
NA = 4
BT = 1024
NT = MAX_T // BT
MAXG = B * NT
NBUF = 8


def _body(g_ref, nlive_ref, bid_ref, tid_ref, s_ref, a_ref, wp_ref, bias_ref,
          out_ref, sbuf, abuf, sem_s, sem_a):
    g = g_ref[0]

    def copies(i, slot):
        b = bid_ref[i]
        t = tid_ref[i]
        c1 = pltpu.make_async_copy(
            s_ref.at[b, pl.ds(t * BT, BT), :], sbuf.at[slot], sem_s.at[slot])
        c2 = pltpu.make_async_copy(
            a_ref.at[b, t], abuf.at[slot], sem_a.at[slot])
        return c1, c2

    for k in range(NBUF - 1):
        c1, c2 = copies(k, k)
        c1.start()
        c2.start()

    def step(i, acc):
        slot = jax.lax.rem(i, NBUF)

        @pl.when(i + NBUF - 1 < g)
        def _():
            c1, c2 = copies(i + NBUF - 1, jax.lax.rem(i + NBUF - 1, NBUF))
            c1.start()
            c2.start()

        c1, c2 = copies(i, slot)
        c1.wait()
        c2.wait()

        x = sbuf[slot]
        z = jnp.dot(x, wp_ref[...], preferred_element_type=jnp.float32)
        zt = z.T[:NA] + bias_ref[...]
        m = jnp.max(zt, axis=0, keepdims=True)
        e = jnp.sum(jnp.exp(zt - m), axis=0, keepdims=True)
        lse = m + jnp.log(e)
        a = abuf[slot]
        taken = jnp.where(a == 0, zt[0:1], 0.0)
        for j in range(1, NA):
            taken += jnp.where(a == j, zt[j:j + 1], 0.0)
        lane = jax.lax.broadcasted_iota(jnp.int32, (1, BT), 1)
        live = lane < nlive_ref[i]
        return acc + jnp.where(live, lse - taken, 0.0)

    out_ref[...] = jax.lax.fori_loop(
        0, g, step, jnp.zeros((1, BT), jnp.float32))


@jax.jit
def _tc_loss(s, actions4, lengths, wp, bias_col):
    lengths = lengths.astype(jnp.int32)
    nblk = (lengths + BT - 1) // BT
    g = jnp.sum(nblk).reshape(1)
    cum = jnp.cumsum(nblk)
    flat = jnp.arange(MAXG, dtype=jnp.int32)
    bid = jnp.searchsorted(cum, flat, side="right").astype(jnp.int32)
    bidc = jnp.minimum(bid, B - 1)
    tid = flat - jnp.where(bid > 0, cum[jnp.maximum(bid - 1, 0)], 0)
    nlive = jnp.clip(lengths[bidc] - tid * BT, 0, BT)

    grid_spec = pltpu.PrefetchScalarGridSpec(
        num_scalar_prefetch=4,
        grid=(1,),
        in_specs=[
            pl.BlockSpec(memory_space=pltpu.MemorySpace.HBM),
            pl.BlockSpec(memory_space=pltpu.MemorySpace.HBM),
            pl.BlockSpec((S_DIM, 8), lambda i, *_: (0, 0)),
            pl.BlockSpec((NA, 1), lambda i, *_: (0, 0)),
        ],
        out_specs=pl.BlockSpec((1, BT), lambda i, *_: (0, 0)),
        scratch_shapes=[
            pltpu.VMEM((NBUF, BT, S_DIM), jnp.float32),
            pltpu.VMEM((NBUF, 1, BT), jnp.int32),
            pltpu.SemaphoreType.DMA((NBUF,)),
            pltpu.SemaphoreType.DMA((NBUF,)),
        ],
    )
    out = pl.pallas_call(
        _body,
        grid_spec=grid_spec,
        out_shape=jax.ShapeDtypeStruct((1, BT), jnp.float32),
    )(g, nlive, bidc, tid, s, actions4, wp, bias_col)
    return jnp.sum(out)


def kernel(s_i_batch, actions_batch, lengths, W, bias, W_stop, W_start):
    del W_stop, W_start
    wp = jnp.zeros((S_DIM, 8), jnp.float32).at[:, :NA].set(W[:, :NA])
    bias_col = bias[:NA].reshape(NA, 1)
    actions4 = actions_batch.astype(jnp.int32).reshape(B, NT, 1, BT)
    return _tc_loss(s_i_batch, actions4, lengths, wp, bias_col)


# speedup vs baseline: 3.7235x; 1.0663x over previous
import jax
import jax.numpy as jnp
from jax.experimental import pallas as pl
from jax.experimental.pallas import tpu as pltpu

B = 16
MAX_T = 4096
S_DIM = 128
NA = 4
BT = 2048
NT = MAX_T // BT
MAXG = B * NT
NBUF = 4


def _body(g_ref, nlive_ref, bid_ref, tid_ref, s_ref, a_ref, wp_ref, bias_ref,
          out_ref, sbuf, abuf, sem_s, sem_a):
    g = g_ref[0]

    def copies(i, slot):
        b = bid_ref[i]
        t = tid_ref[i]
        c1 = pltpu.make_async_copy(
            s_ref.at[b, pl.ds(t * BT, BT), :], sbuf.at[slot], sem_s.at[slot])
        c2 = pltpu.make_async_copy(
            a_ref.at[b, t], abuf.at[slot], sem_a.at[slot])
        return c1, c2

    for k in range(NBUF - 1):
        c1, c2 = copies(k, k)
        c1.start()
        c2.start()

    def step(i, acc):
        slot = jax.lax.rem(i, NBUF)

        @pl.when(i + NBUF - 1 < g)
        def _():
            c1, c2 = copies(i + NBUF - 1, jax.lax.rem(i + NBUF - 1, NBUF))
            c1.start()
            c2.start()

        c1, c2 = copies(i, slot)
        c1.wait()
        c2.wait()

        x = sbuf[slot]
        z = jnp.dot(x, wp_ref[...], preferred_element_type=jnp.float32)
        zt = z.T[:NA] + bias_ref[...]
        m = jnp.max(zt, axis=0, keepdims=True)
        e = jnp.sum(jnp.exp(zt - m), axis=0, keepdims=True)
        lse = m + jnp.log(e)
        a = abuf[slot]
        taken = jnp.where(a == 0, zt[0:1], 0.0)
        for j in range(1, NA):
            taken += jnp.where(a == j, zt[j:j + 1], 0.0)
        lane = jax.lax.broadcasted_iota(jnp.int32, (1, BT), 1)
        live = lane < nlive_ref[i]
        return acc + jnp.where(live, lse - taken, 0.0)

    out_ref[...] = jax.lax.fori_loop(
        0, g, step, jnp.zeros((1, BT), jnp.float32))


@jax.jit
def _tc_loss(s, actions4, lengths, wp, bias_col):
    lengths = lengths.astype(jnp.int32)
    nblk = (lengths + BT - 1) // BT
    g = jnp.sum(nblk).reshape(1)
    cum = jnp.cumsum(nblk)
    flat = jnp.arange(MAXG, dtype=jnp.int32)
    bid = jnp.searchsorted(cum, flat, side="right").astype(jnp.int32)
    bidc = jnp.minimum(bid, B - 1)
    tid = flat - jnp.where(bid > 0, cum[jnp.maximum(bid - 1, 0)], 0)
    nlive = jnp.clip(lengths[bidc] - tid * BT, 0, BT)

    grid_spec = pltpu.PrefetchScalarGridSpec(
        num_scalar_prefetch=4,
        grid=(1,),
        in_specs=[
            pl.BlockSpec(memory_space=pltpu.MemorySpace.HBM),
            pl.BlockSpec(memory_space=pltpu.MemorySpace.HBM),
            pl.BlockSpec((S_DIM, 8), lambda i, *_: (0, 0)),
            pl.BlockSpec((NA, 1), lambda i, *_: (0, 0)),
        ],
        out_specs=pl.BlockSpec((1, BT), lambda i, *_: (0, 0)),
        scratch_shapes=[
            pltpu.VMEM((NBUF, BT, S_DIM), jnp.float32),
            pltpu.VMEM((NBUF, 1, BT), jnp.int32),
            pltpu.SemaphoreType.DMA((NBUF,)),
            pltpu.SemaphoreType.DMA((NBUF,)),
        ],
    )
    out = pl.pallas_call(
        _body,
        grid_spec=grid_spec,
        out_shape=jax.ShapeDtypeStruct((1, BT), jnp.float32),
    )(g, nlive, bidc, tid, s, actions4, wp, bias_col)
    return jnp.sum(out)


def kernel(s_i_batch, actions_batch, lengths, W, bias, W_stop, W_start):
    del W_stop, W_start
    wp = jnp.zeros((S_DIM, 8), jnp.float32).at[:, :NA].set(W[:, :NA])
    bias_col = bias[:NA].reshape(NA, 1)
    actions4 = actions_batch.astype(jnp.int32).reshape(B, NT, 1, BT)
    return _tc_loss(s_i_batch, actions4, lengths, wp, bias_col)
